# baseline (speedup 1.0000x reference)
import functools

import jax
import jax.numpy as jnp
from jax import lax
from jax.experimental import pallas as pl
from jax.experimental.pallas import tpu as pltpu
from jax.experimental.pallas import tpu_sc as plsc

_N = 10000
_E = 320000
_DIN = 128
_DOUT = 64
_B = 64

_NC = 2
_NS = 16
_NW = _NC * _NS
_EPW = _E // _NW
_CK = 256
_NCH = 40
_EP = _NW * _CK * _NCH
_RPT = 632
_SEGS = ((0, 248), (248, 248), (496, 136))

_mesh = plsc.VectorSubcoreMesh(core_axis_name="c", subcore_axis_name="s")
_sc_params = pltpu.CompilerParams(use_tc_tiling_on_sc=False,
                                  needs_layout_passes=False)


@functools.partial(
    pl.kernel,
    out_type=[
        jax.ShapeDtypeStruct((_NC, _N), jnp.float32),
        jax.ShapeDtypeStruct((_NC, _N), jnp.float32),
    ],
    mesh=_mesh,
    scratch_types=[
        pltpu.VMEM((_EPW,), jnp.int32),
        pltpu.VMEM((_EPW,), jnp.int32),
        pltpu.VMEM((_EPW,), jnp.float32),
        pltpu.VMEM_SHARED((_N,), jnp.float32),
        pltpu.VMEM_SHARED((_N,), jnp.float32),
    ],
)
def _deg_kernel(src_hbm, dst_hbm, dego_hbm, degi_hbm,
                srcv, dstv, onesv, dego_s, degi_s):
    cid = lax.axis_index("c")
    sid = lax.axis_index("s")
    base = (cid * _NS + sid) * _EPW

    def zloop(i, _):
        onesv[pl.ds(i * 16, 16)] = jnp.zeros((16,), jnp.float32)
        return 0
    lax.fori_loop(0, _EPW // 16, zloop, 0)

    @pl.when(sid == 0)
    def _():
        pltpu.sync_copy(onesv, dego_s)
        pltpu.sync_copy(onesv, degi_s)

    def oloop(i, _):
        onesv[pl.ds(i * 16, 16)] = jnp.ones((16,), jnp.float32)
        return 0
    lax.fori_loop(0, _EPW // 16, oloop, 0)

    plsc.subcore_barrier()
    pltpu.sync_copy(src_hbm.at[pl.ds(base, _EPW)], srcv)
    pltpu.sync_copy(dst_hbm.at[pl.ds(base, _EPW)], dstv)
    pltpu.sync_copy(onesv, dego_s.at[srcv], add=True)
    pltpu.sync_copy(onesv, degi_s.at[dstv], add=True)
    plsc.subcore_barrier()

    @pl.when(sid == 0)
    def _():
        pltpu.sync_copy(dego_s, dego_hbm.at[cid])
        pltpu.sync_copy(degi_s, degi_hbm.at[cid])


@functools.partial(
    pl.kernel,
    out_type=jax.ShapeDtypeStruct((_NC * _N, _DOUT), jnp.float32),
    mesh=_mesh,
    scratch_types=[
        pltpu.VMEM((_NCH, _CK), jnp.int32),
        pltpu.VMEM((_NCH, _CK), jnp.int32),
        pltpu.VMEM((_NCH, _CK), jnp.float32),
        pltpu.VMEM((_CK, _DOUT), jnp.float32),
        pltpu.VMEM((_CK, _DOUT), jnp.float32),
        pltpu.VMEM((_N,), jnp.float32),
        pltpu.VMEM((2000,), jnp.float32),
        pltpu.VMEM((_CK,), jnp.float32),
        pltpu.VMEM_SHARED((_N, _DOUT), jnp.float32),
        pltpu.SemaphoreType.DMA,
        pltpu.SemaphoreType.DMA,
        pltpu.SemaphoreType.DMA,
        pltpu.SemaphoreType.DMA,
    ],
    compiler_params=_sc_params,
)
def _edge_kernel(xw_hbm, dego_hbm, src_hbm, dst_hbm, ew_hbm, agg_hbm,
                 srcb, dstb, ewb, rows0, rows1, normv, tmpv, wbuf, agg_s,
                 semg0, semg1, sems0, sems1):
    cid = lax.axis_index("c")
    sid = lax.axis_index("s")
    r0 = (cid * _NS + sid) * _NCH
    pltpu.sync_copy(src_hbm.at[pl.ds(r0, _NCH)], srcb)
    pltpu.sync_copy(dst_hbm.at[pl.ds(r0, _NCH)], dstb)
    pltpu.sync_copy(ew_hbm.at[pl.ds(r0, _NCH)], ewb)

    def zloop(i, _):
        for c in range(_DOUT // 16):
            rows0[i, pl.ds(c * 16, 16)] = jnp.zeros((16,), jnp.float32)
        return 0
    lax.fori_loop(0, _CK, zloop, 0)
    row0 = pl.multiple_of(jnp.minimum(sid * _RPT, _N - _RPT), 8)
    for o, ln in _SEGS:
        pltpu.sync_copy(rows0.at[pl.ds(0, ln)], agg_s.at[pl.ds(row0 + o, ln)])
    pltpu.async_copy(xw_hbm.at[srcb.at[0]], rows0, semg0)

    pltpu.sync_copy(dego_hbm.at[0], normv)

    def nadd(s, _):
        pltpu.sync_copy(dego_hbm.at[1, pl.ds(s * 2000, 2000)], tmpv)

        def aloop(q, _):
            sl = pl.ds(q * 16, 16)
            d = normv[pl.ds(s * 2000 + q * 16, 16)] + tmpv[sl]
            x = jnp.maximum(d, 1.0)
            i32 = plsc.bitcast(x, jnp.int32)
            y = plsc.bitcast(
                jnp.full((16,), 0x5F3759DF, jnp.int32)
                - lax.shift_right_logical(i32, 1), jnp.float32)
            hx = 0.5 * x
            for _it in range(3):
                y = y * (1.5 - hx * y * y)
            normv[pl.ds(s * 2000 + q * 16, 16)] = y
            return 0
        lax.fori_loop(0, 125, aloop, 0)
        return 0
    lax.fori_loop(0, _N // 2000, nadd, 0)
    plsc.subcore_barrier()

    def scale(rows, j):
        jv = jnp.full((16,), j, jnp.int32)

        def wloop(q, _):
            sl = pl.ds(q * 16, 16)
            sv = srcb[j, sl]
            nv = plsc.load_gather(normv, [sv])
            wbuf[sl] = ewb[j, sl] * nv
            return 0
        lax.fori_loop(0, _CK // 16, wloop, 0)

        def srow(q, _):
            for u in range(2):
                i = q * 2 + u
                wv = plsc.load_gather(wbuf, [jnp.full((16,), i, jnp.int32)])
                for c in range(_DOUT // 16):
                    sl = pl.ds(c * 16, 16)
                    rows[i, sl] = rows[i, sl] * wv
            return 0
        lax.fori_loop(0, _CK // 2, srow, 0)

    def body(g, _):
        c0 = 2 * g
        c1 = c0 + 1
        @pl.when(g > 0)
        def _():
            pltpu.make_async_copy(rows1, agg_s.at[dstb.at[c1]], sems1).wait()
        pltpu.async_copy(xw_hbm.at[srcb.at[c1]], rows1, semg1)
        pltpu.make_async_copy(xw_hbm.at[srcb.at[c0]], rows0, semg0).wait()
        scale(rows0, c0)
        pltpu.async_copy(rows0, agg_s.at[dstb.at[c0]], sems0, add=True)
        pltpu.make_async_copy(xw_hbm.at[srcb.at[c1]], rows1, semg1).wait()
        scale(rows1, c1)
        pltpu.async_copy(rows1, agg_s.at[dstb.at[c1]], sems1, add=True)
        pltpu.make_async_copy(rows0, agg_s.at[dstb.at[c0]], sems0).wait()
        @pl.when(g < _NCH // 2 - 1)
        def _():
            pltpu.async_copy(xw_hbm.at[srcb.at[c0 + 2]], rows0, semg0)
        return 0
    lax.fori_loop(0, _NCH // 2, body, 0)
    pltpu.make_async_copy(rows1, agg_s.at[dstb.at[_NCH - 1]], sems1).wait()

    plsc.subcore_barrier()
    for o, ln in _SEGS:
        pltpu.sync_copy(agg_s.at[pl.ds(row0 + o, ln)],
                        agg_hbm.at[pl.ds(cid * _N + row0 + o, ln)])


def _mm_body(x_ref, w_ref, out_ref):
    out_ref[...] = jnp.dot(x_ref[...], w_ref[...],
                           preferred_element_type=jnp.float32)


_mm_call = pl.pallas_call(
    _mm_body,
    out_shape=jax.ShapeDtypeStruct((_N, _DOUT), jnp.float32),
)


def _epi_body(x_ref, w_ref, agg_ref, degi_ref, n2g_ref, b_ref, a_ref,
              h_out, pool_out, anc_out):
    a = a_ref[0, 0]
    bias = b_ref[...]
    agg = agg_ref[pl.ds(0, _N), :] + agg_ref[pl.ds(_N, _N), :]
    degi = degi_ref[0] + degi_ref[1]
    ndst = lax.rsqrt(jnp.maximum(degi, 1.0))
    h = agg * ndst + bias
    hp = jnp.maximum(h, 0.0) + a * jnp.minimum(h, 0.0)
    hn = jnp.sqrt(jnp.sum(hp * hp, axis=1, keepdims=True))
    h_out[...] = hp / jnp.maximum(hn, 1e-12)

    n2g = n2g_ref[...]
    gids = lax.broadcasted_iota(jnp.int32, (_N, _B), 1)
    oh = (n2g == gids).astype(jnp.float32)
    ones_col = jnp.ones((_N, 1), jnp.float32)
    cdims = (((0,), (0,)), ((), ()))
    pool_sum = lax.dot_general(oh, hp, cdims, preferred_element_type=jnp.float32)
    cnt = lax.dot_general(oh, ones_col, cdims, preferred_element_type=jnp.float32)
    pool = pool_sum / jnp.maximum(cnt, 1.0)
    pn = jnp.sqrt(jnp.sum(pool * pool, axis=1, keepdims=True))
    pool_out[...] = pool / jnp.maximum(pn, 1e-12)

    less = (n2g < gids).astype(jnp.float32)
    cntl = lax.dot_general(ones_col, less, cdims, preferred_element_type=jnp.float32)
    aidx = jnp.minimum(cntl, float(_N - 1)).astype(jnp.int32)
    nio = lax.broadcasted_iota(jnp.int32, (_N, _B), 0)
    aoh = (nio == aidx).astype(jnp.float32)
    ax = lax.dot_general(aoh, x_ref[...], cdims, preferred_element_type=jnp.float32)
    ao = jnp.dot(ax, w_ref[...], preferred_element_type=jnp.float32) + bias
    aop = jnp.maximum(ao, 0.0) + a * jnp.minimum(ao, 0.0)
    an = jnp.sqrt(jnp.sum(aop * aop, axis=1, keepdims=True))
    anc_out[...] = aop / jnp.maximum(an, 1e-12)


_epi_call = pl.pallas_call(
    _epi_body,
    out_shape=[
        jax.ShapeDtypeStruct((_N, _DOUT), jnp.float32),
        jax.ShapeDtypeStruct((_B, _DOUT), jnp.float32),
        jax.ShapeDtypeStruct((_B, _DOUT), jnp.float32),
    ],
)


def kernel(x, edge_index, edge_weight, node2graph, W, b, prelu_a):
    src = edge_index[0]
    dst = edge_index[1]
    xw = _mm_call(x, W)
    dego, degi = _deg_kernel(src, dst)
    pad = _EP - _E
    srcp = jnp.concatenate([src, jnp.zeros((pad,), jnp.int32)])
    dstp = jnp.concatenate([dst, jnp.zeros((pad,), jnp.int32)])
    ewp = jnp.concatenate([edge_weight, jnp.zeros((pad,), jnp.float32)])
    agg2 = _edge_kernel(xw, dego,
                        srcp.reshape(_EP // _CK, _CK),
                        dstp.reshape(_EP // _CK, _CK),
                        ewp.reshape(_EP // _CK, _CK))
    h, pool, anc = _epi_call(x, W, agg2, degi.reshape(_NC, _N, 1),
                             node2graph.reshape(_N, 1),
                             b.reshape(1, _DOUT),
                             jnp.asarray(prelu_a, jnp.float32).reshape(1, 1))
    return h, pool, anc


# speedup vs baseline: 1.8553x; 1.8553x over previous
import functools

import jax
import jax.numpy as jnp
from jax import lax
from jax.experimental import pallas as pl
from jax.experimental.pallas import tpu as pltpu
from jax.experimental.pallas import tpu_sc as plsc

_N = 10000
_E = 320000
_DIN = 128
_DOUT = 64
_B = 64

_NC = 2
_NS = 16
_NW = _NC * _NS
_EPW = _E // _NW
_CK = 256
_NCH = 40
_EP = _NW * _CK * _NCH
_RPT = 632
_SEGS = ((0, 248), (248, 248), (496, 136))

_mesh = plsc.VectorSubcoreMesh(core_axis_name="c", subcore_axis_name="s")
_sc_params = pltpu.CompilerParams(use_tc_tiling_on_sc=False,
                                  needs_layout_passes=False)


@functools.partial(
    pl.kernel,
    out_type=[
        jax.ShapeDtypeStruct((_NC, _N), jnp.float32),
        jax.ShapeDtypeStruct((_NC, _N), jnp.float32),
    ],
    mesh=_mesh,
    scratch_types=[
        pltpu.VMEM((_EPW,), jnp.int32),
        pltpu.VMEM((_EPW,), jnp.int32),
        pltpu.VMEM((_EPW,), jnp.float32),
        pltpu.VMEM_SHARED((_N,), jnp.float32),
        pltpu.VMEM_SHARED((_N,), jnp.float32),
    ],
)
def _deg_kernel(src_hbm, dst_hbm, dego_hbm, degi_hbm,
                srcv, dstv, onesv, dego_s, degi_s):
    cid = lax.axis_index("c")
    sid = lax.axis_index("s")
    base = (cid * _NS + sid) * _EPW

    def zloop(i, _):
        onesv[pl.ds(i * 16, 16)] = jnp.zeros((16,), jnp.float32)
        return 0
    lax.fori_loop(0, _EPW // 16, zloop, 0)

    @pl.when(sid == 0)
    def _():
        pltpu.sync_copy(onesv, dego_s)
        pltpu.sync_copy(onesv, degi_s)

    def oloop(i, _):
        onesv[pl.ds(i * 16, 16)] = jnp.ones((16,), jnp.float32)
        return 0
    lax.fori_loop(0, _EPW // 16, oloop, 0)

    plsc.subcore_barrier()
    pltpu.sync_copy(src_hbm.at[pl.ds(base, _EPW)], srcv)
    pltpu.sync_copy(dst_hbm.at[pl.ds(base, _EPW)], dstv)
    pltpu.sync_copy(onesv, dego_s.at[srcv], add=True)
    pltpu.sync_copy(onesv, degi_s.at[dstv], add=True)
    plsc.subcore_barrier()

    @pl.when(sid == 0)
    def _():
        pltpu.sync_copy(dego_s, dego_hbm.at[cid])
        pltpu.sync_copy(degi_s, degi_hbm.at[cid])


@functools.partial(
    pl.kernel,
    out_type=jax.ShapeDtypeStruct((_NC * _N, _DOUT), jnp.float32),
    mesh=_mesh,
    scratch_types=[
        pltpu.VMEM((_NCH, _CK), jnp.int32),
        pltpu.VMEM((_NCH, _CK), jnp.int32),
        pltpu.VMEM((_NCH, _CK), jnp.float32),
        pltpu.VMEM((_CK, _DOUT), jnp.float32),
        pltpu.VMEM((_CK, _DOUT), jnp.float32),
        pltpu.VMEM((_N,), jnp.float32),
        pltpu.VMEM((640,), jnp.float32),
        pltpu.VMEM((640,), jnp.float32),
        pltpu.VMEM((_CK,), jnp.float32),
        pltpu.VMEM_SHARED((_N,), jnp.float32),
        pltpu.VMEM_SHARED((_N, _DOUT), jnp.float32),
        pltpu.SemaphoreType.DMA,
        pltpu.SemaphoreType.DMA,
        pltpu.SemaphoreType.DMA,
        pltpu.SemaphoreType.DMA,
    ],
    compiler_params=_sc_params,
)
def _edge_kernel(xw_hbm, dego_hbm, src_hbm, dst_hbm, ew_hbm, agg_hbm,
                 srcb, dstb, ewb, rows0, rows1, normv, na, nb, wbuf,
                 norm_s, agg_s, semg0, semg1, sems0, sems1):
    cid = lax.axis_index("c")
    sid = lax.axis_index("s")
    r0 = (cid * _NS + sid) * _NCH
    pltpu.sync_copy(src_hbm.at[pl.ds(r0, _NCH)], srcb)
    pltpu.sync_copy(dst_hbm.at[pl.ds(r0, _NCH)], dstb)
    pltpu.sync_copy(ew_hbm.at[pl.ds(r0, _NCH)], ewb)

    def zloop(i, _):
        for c in range(_DOUT // 16):
            rows0[i, pl.ds(c * 16, 16)] = jnp.zeros((16,), jnp.float32)
        return 0
    lax.fori_loop(0, _CK, zloop, 0)
    row0 = pl.multiple_of(jnp.minimum(sid * _RPT, _N - _RPT), 8)
    for o, ln in _SEGS:
        pltpu.sync_copy(rows0.at[pl.ds(0, ln)], agg_s.at[pl.ds(row0 + o, ln)])
    pltpu.async_copy(xw_hbm.at[srcb.at[0]], rows0, semg0)

    offn = pl.multiple_of(sid * 624, 16)
    pltpu.sync_copy(dego_hbm.at[0, pl.ds(offn, 640)], na)
    pltpu.sync_copy(dego_hbm.at[1, pl.ds(offn, 640)], nb)

    def nloop(q, _):
        sl = pl.ds(q * 16, 16)
        x = jnp.maximum(na[sl] + nb[sl], 1.0)
        i32 = plsc.bitcast(x, jnp.int32)
        y = plsc.bitcast(
            jnp.full((16,), 0x5F3759DF, jnp.int32)
            - lax.shift_right_logical(i32, 1), jnp.float32)
        hx = 0.5 * x
        for _it in range(3):
            y = y * (1.5 - hx * y * y)
        na[sl] = y
        return 0
    lax.fori_loop(0, 40, nloop, 0)
    pltpu.sync_copy(na, norm_s.at[pl.ds(offn, 640)])
    plsc.subcore_barrier()
    pltpu.sync_copy(norm_s, normv)

    def scale(rows, j):
        jv = jnp.full((16,), j, jnp.int32)

        def wloop(q, _):
            sl = pl.ds(q * 16, 16)
            sv = srcb[j, sl]
            nv = plsc.load_gather(normv, [sv])
            wbuf[sl] = ewb[j, sl] * nv
            return 0
        lax.fori_loop(0, _CK // 16, wloop, 0)

        def srow(q, _):
            for u in range(2):
                i = q * 2 + u
                wv = plsc.load_gather(wbuf, [jnp.full((16,), i, jnp.int32)])
                for c in range(_DOUT // 16):
                    sl = pl.ds(c * 16, 16)
                    rows[i, sl] = rows[i, sl] * wv
            return 0
        lax.fori_loop(0, _CK // 2, srow, 0)

    def body(g, _):
        c0 = 2 * g
        c1 = c0 + 1
        @pl.when(g > 0)
        def _():
            pltpu.make_async_copy(rows1, agg_s.at[dstb.at[c1]], sems1).wait()
        pltpu.async_copy(xw_hbm.at[srcb.at[c1]], rows1, semg1)
        pltpu.make_async_copy(xw_hbm.at[srcb.at[c0]], rows0, semg0).wait()
        scale(rows0, c0)
        pltpu.async_copy(rows0, agg_s.at[dstb.at[c0]], sems0, add=True)
        pltpu.make_async_copy(xw_hbm.at[srcb.at[c1]], rows1, semg1).wait()
        scale(rows1, c1)
        pltpu.async_copy(rows1, agg_s.at[dstb.at[c1]], sems1, add=True)
        pltpu.make_async_copy(rows0, agg_s.at[dstb.at[c0]], sems0).wait()
        @pl.when(g < _NCH // 2 - 1)
        def _():
            pltpu.async_copy(xw_hbm.at[srcb.at[c0 + 2]], rows0, semg0)
        return 0
    lax.fori_loop(0, _NCH // 2, body, 0)
    pltpu.make_async_copy(rows1, agg_s.at[dstb.at[_NCH - 1]], sems1).wait()

    plsc.subcore_barrier()
    for o, ln in _SEGS:
        pltpu.sync_copy(agg_s.at[pl.ds(row0 + o, ln)],
                        agg_hbm.at[pl.ds(cid * _N + row0 + o, ln)])


def _mm_body(x_ref, w_ref, out_ref):
    out_ref[...] = jnp.dot(x_ref[...], w_ref[...],
                           preferred_element_type=jnp.float32)


_mm_call = pl.pallas_call(
    _mm_body,
    out_shape=jax.ShapeDtypeStruct((_N, _DOUT), jnp.float32),
)


def _epi_body(x_ref, w_ref, agg_ref, degi_ref, n2g_ref, b_ref, a_ref,
              h_out, pool_out, anc_out):
    a = a_ref[0, 0]
    bias = b_ref[...]
    agg = agg_ref[pl.ds(0, _N), :] + agg_ref[pl.ds(_N, _N), :]
    degi = degi_ref[0] + degi_ref[1]
    ndst = lax.rsqrt(jnp.maximum(degi, 1.0))
    h = agg * ndst + bias
    hp = jnp.maximum(h, 0.0) + a * jnp.minimum(h, 0.0)
    hn = jnp.sqrt(jnp.sum(hp * hp, axis=1, keepdims=True))
    h_out[...] = hp / jnp.maximum(hn, 1e-12)

    n2g = n2g_ref[...]
    gids = lax.broadcasted_iota(jnp.int32, (_N, _B), 1)
    oh = (n2g == gids).astype(jnp.float32)
    ones_col = jnp.ones((_N, 1), jnp.float32)
    cdims = (((0,), (0,)), ((), ()))
    pool_sum = lax.dot_general(oh, hp, cdims, preferred_element_type=jnp.float32)
    cnt = lax.dot_general(oh, ones_col, cdims, preferred_element_type=jnp.float32)
    pool = pool_sum / jnp.maximum(cnt, 1.0)
    pn = jnp.sqrt(jnp.sum(pool * pool, axis=1, keepdims=True))
    pool_out[...] = pool / jnp.maximum(pn, 1e-12)

    less = (n2g < gids).astype(jnp.float32)
    cntl = lax.dot_general(ones_col, less, cdims, preferred_element_type=jnp.float32)
    aidx = jnp.minimum(cntl, float(_N - 1)).astype(jnp.int32)
    nio = lax.broadcasted_iota(jnp.int32, (_N, _B), 0)
    aoh = (nio == aidx).astype(jnp.float32)
    ax = lax.dot_general(aoh, x_ref[...], cdims, preferred_element_type=jnp.float32)
    ao = jnp.dot(ax, w_ref[...], preferred_element_type=jnp.float32) + bias
    aop = jnp.maximum(ao, 0.0) + a * jnp.minimum(ao, 0.0)
    an = jnp.sqrt(jnp.sum(aop * aop, axis=1, keepdims=True))
    anc_out[...] = aop / jnp.maximum(an, 1e-12)


_epi_call = pl.pallas_call(
    _epi_body,
    out_shape=[
        jax.ShapeDtypeStruct((_N, _DOUT), jnp.float32),
        jax.ShapeDtypeStruct((_B, _DOUT), jnp.float32),
        jax.ShapeDtypeStruct((_B, _DOUT), jnp.float32),
    ],
)


def kernel(x, edge_index, edge_weight, node2graph, W, b, prelu_a):
    src = edge_index[0]
    dst = edge_index[1]
    xw = _mm_call(x, W)
    dego, degi = _deg_kernel(src, dst)
    pad = _EP - _E
    spread = jnp.arange(pad, dtype=jnp.int32) % _N
    srcp = jnp.concatenate([src, spread])
    dstp = jnp.concatenate([dst, spread])
    ewp = jnp.concatenate([edge_weight, jnp.zeros((pad,), jnp.float32)])
    agg2 = _edge_kernel(xw, dego,
                        srcp.reshape(_EP // _CK, _CK),
                        dstp.reshape(_EP // _CK, _CK),
                        ewp.reshape(_EP // _CK, _CK))
    h, pool, anc = _epi_call(x, W, agg2, degi.reshape(_NC, _N, 1),
                             node2graph.reshape(_N, 1),
                             b.reshape(1, _DOUT),
                             jnp.asarray(prelu_a, jnp.float32).reshape(1, 1))
    return h, pool, anc


# speedup vs baseline: 2.3142x; 1.2473x over previous
import functools

import jax
import jax.numpy as jnp
from jax import lax
from jax.experimental import pallas as pl
from jax.experimental.pallas import tpu as pltpu
from jax.experimental.pallas import tpu_sc as plsc

_N = 10000
_E = 320000
_DIN = 128
_DOUT = 64
_B = 64

_NC = 2
_NS = 16
_NW = _NC * _NS
_EPW = _E // _NW
_CK = 256
_NCH = 40
_EP = _NW * _CK * _NCH
_RPT = 632
_SEGS = ((0, 248), (248, 248), (496, 136))

_mesh = plsc.VectorSubcoreMesh(core_axis_name="c", subcore_axis_name="s")
_sc_params = pltpu.CompilerParams(use_tc_tiling_on_sc=False,
                                  needs_layout_passes=False)


@functools.partial(
    pl.kernel,
    out_type=[
        jax.ShapeDtypeStruct((_NC, _N), jnp.float32),
        jax.ShapeDtypeStruct((_NC, _N), jnp.float32),
    ],
    mesh=_mesh,
    scratch_types=[
        pltpu.VMEM((_EPW,), jnp.int32),
        pltpu.VMEM((_EPW,), jnp.int32),
        pltpu.VMEM((_EPW,), jnp.float32),
        pltpu.VMEM_SHARED((_N,), jnp.float32),
        pltpu.VMEM_SHARED((_N,), jnp.float32),
    ],
)
def _deg_kernel(src_hbm, dst_hbm, dego_hbm, degi_hbm,
                srcv, dstv, onesv, dego_s, degi_s):
    cid = lax.axis_index("c")
    sid = lax.axis_index("s")
    base = (cid * _NS + sid) * _EPW

    def zloop(i, _):
        onesv[pl.ds(i * 16, 16)] = jnp.zeros((16,), jnp.float32)
        return 0
    lax.fori_loop(0, _EPW // 16, zloop, 0)

    @pl.when(sid == 0)
    def _():
        pltpu.sync_copy(onesv, dego_s)
        pltpu.sync_copy(onesv, degi_s)

    def oloop(i, _):
        onesv[pl.ds(i * 16, 16)] = jnp.ones((16,), jnp.float32)
        return 0
    lax.fori_loop(0, _EPW // 16, oloop, 0)

    plsc.subcore_barrier()
    pltpu.sync_copy(src_hbm.at[pl.ds(base, _EPW)], srcv)
    pltpu.sync_copy(dst_hbm.at[pl.ds(base, _EPW)], dstv)
    pltpu.sync_copy(onesv, dego_s.at[srcv], add=True)
    pltpu.sync_copy(onesv, degi_s.at[dstv], add=True)
    plsc.subcore_barrier()

    @pl.when(sid == 0)
    def _():
        pltpu.sync_copy(dego_s, dego_hbm.at[cid])
        pltpu.sync_copy(degi_s, degi_hbm.at[cid])


@functools.partial(
    pl.kernel,
    out_type=jax.ShapeDtypeStruct((_NC * _N, _DOUT), jnp.float32),
    mesh=_mesh,
    scratch_types=[
        pltpu.VMEM((_NCH, _CK), jnp.int32),
        pltpu.VMEM((_NCH, _CK), jnp.int32),
        pltpu.VMEM((_NCH, _CK), jnp.float32),
        pltpu.VMEM((_CK, _DOUT), jnp.float32),
        pltpu.VMEM((_CK, _DOUT), jnp.float32),
        pltpu.VMEM((_N,), jnp.float32),
        pltpu.VMEM((640,), jnp.float32),
        pltpu.VMEM((640,), jnp.float32),
        pltpu.VMEM((_CK,), jnp.float32),
        pltpu.VMEM_SHARED((_N,), jnp.float32),
        pltpu.VMEM_SHARED((_N, _DOUT), jnp.float32),
        pltpu.SemaphoreType.DMA,
        pltpu.SemaphoreType.DMA,
        pltpu.SemaphoreType.DMA,
        pltpu.SemaphoreType.DMA,
    ],
    compiler_params=_sc_params,
)
def _edge_kernel(xw_hbm, dego_hbm, src_hbm, dst_hbm, ew_hbm, agg_hbm,
                 srcb, dstb, ewb, rows0, rows1, normv, na, nb, wbuf,
                 norm_s, agg_s, semg0, semg1, sems0, sems1):
    cid = lax.axis_index("c")
    sid = lax.axis_index("s")
    r0 = (cid * _NS + sid) * _NCH
    pltpu.sync_copy(src_hbm.at[pl.ds(r0, _NCH)], srcb)
    pltpu.sync_copy(dst_hbm.at[pl.ds(r0, _NCH)], dstb)
    pltpu.sync_copy(ew_hbm.at[pl.ds(r0, _NCH)], ewb)

    def zloop(i, _):
        for c in range(_DOUT // 16):
            rows0[i, pl.ds(c * 16, 16)] = jnp.zeros((16,), jnp.float32)
        return 0
    lax.fori_loop(0, _CK, zloop, 0)
    row0 = pl.multiple_of(jnp.minimum(sid * _RPT, _N - _RPT), 8)
    for o, ln in _SEGS:
        pltpu.sync_copy(rows0.at[pl.ds(0, ln)], agg_s.at[pl.ds(row0 + o, ln)])
    pltpu.async_copy(xw_hbm.at[srcb.at[0]], rows0, semg0)

    offn = pl.multiple_of(sid * 624, 16)
    pltpu.sync_copy(dego_hbm.at[0, pl.ds(offn, 640)], na)
    pltpu.sync_copy(dego_hbm.at[1, pl.ds(offn, 640)], nb)

    def nloop(q, _):
        sl = pl.ds(q * 16, 16)
        x = jnp.maximum(na[sl] + nb[sl], 1.0)
        i32 = plsc.bitcast(x, jnp.int32)
        y = plsc.bitcast(
            jnp.full((16,), 0x5F3759DF, jnp.int32)
            - lax.shift_right_logical(i32, 1), jnp.float32)
        hx = 0.5 * x
        for _it in range(3):
            y = y * (1.5 - hx * y * y)
        na[sl] = y
        return 0
    lax.fori_loop(0, 40, nloop, 0)
    pltpu.sync_copy(na, norm_s.at[pl.ds(offn, 640)])
    plsc.subcore_barrier()
    pltpu.sync_copy(norm_s, normv)

    def scale(rows, j):
        jv = jnp.full((16,), j, jnp.int32)

        def wloop(q, _):
            sl = pl.ds(q * 16, 16)
            sv = srcb[j, sl]
            nv = plsc.load_gather(normv, [sv])
            wbuf[sl] = ewb[j, sl] * nv
            return 0
        lax.fori_loop(0, _CK // 16, wloop, 0)

        @plsc.parallel_loop(0, _CK, unroll=8)
        def srow(i):
            wv = plsc.load_gather(wbuf, [jnp.full((16,), i, jnp.int32)])
            for c in range(_DOUT // 16):
                sl = pl.ds(c * 16, 16)
                rows[i, sl] = rows[i, sl] * wv

    def body(g, _):
        c0 = 2 * g
        c1 = c0 + 1
        @pl.when(g > 0)
        def _():
            pltpu.make_async_copy(rows1, agg_s.at[dstb.at[c1]], sems1).wait()
        pltpu.async_copy(xw_hbm.at[srcb.at[c1]], rows1, semg1)
        pltpu.make_async_copy(xw_hbm.at[srcb.at[c0]], rows0, semg0).wait()
        scale(rows0, c0)
        pltpu.async_copy(rows0, agg_s.at[dstb.at[c0]], sems0, add=True)
        pltpu.make_async_copy(xw_hbm.at[srcb.at[c1]], rows1, semg1).wait()
        scale(rows1, c1)
        pltpu.async_copy(rows1, agg_s.at[dstb.at[c1]], sems1, add=True)
        pltpu.make_async_copy(rows0, agg_s.at[dstb.at[c0]], sems0).wait()
        @pl.when(g < _NCH // 2 - 1)
        def _():
            pltpu.async_copy(xw_hbm.at[srcb.at[c0 + 2]], rows0, semg0)
        return 0
    lax.fori_loop(0, _NCH // 2, body, 0)
    pltpu.make_async_copy(rows1, agg_s.at[dstb.at[_NCH - 1]], sems1).wait()

    plsc.subcore_barrier()
    for o, ln in _SEGS:
        pltpu.sync_copy(agg_s.at[pl.ds(row0 + o, ln)],
                        agg_hbm.at[pl.ds(cid * _N + row0 + o, ln)])


def _mm_body(x_ref, w_ref, out_ref):
    out_ref[...] = jnp.dot(x_ref[...], w_ref[...],
                           preferred_element_type=jnp.float32)


_mm_call = pl.pallas_call(
    _mm_body,
    out_shape=jax.ShapeDtypeStruct((_N, _DOUT), jnp.float32),
)


def _epi_body(x_ref, w_ref, agg_ref, degi_ref, n2g_ref, b_ref, a_ref,
              h_out, pool_out, anc_out):
    a = a_ref[0, 0]
    bias = b_ref[...]
    agg = agg_ref[pl.ds(0, _N), :] + agg_ref[pl.ds(_N, _N), :]
    degi = degi_ref[0] + degi_ref[1]
    ndst = lax.rsqrt(jnp.maximum(degi, 1.0))
    h = agg * ndst + bias
    hp = jnp.maximum(h, 0.0) + a * jnp.minimum(h, 0.0)
    hn = jnp.sqrt(jnp.sum(hp * hp, axis=1, keepdims=True))
    h_out[...] = hp / jnp.maximum(hn, 1e-12)

    n2g = n2g_ref[...]
    gids = lax.broadcasted_iota(jnp.int32, (_N, _B), 1)
    oh = (n2g == gids).astype(jnp.float32)
    ones_col = jnp.ones((_N, 1), jnp.float32)
    cdims = (((0,), (0,)), ((), ()))
    pool_sum = lax.dot_general(oh, hp, cdims, preferred_element_type=jnp.float32)
    cnt = lax.dot_general(oh, ones_col, cdims, preferred_element_type=jnp.float32)
    pool = pool_sum / jnp.maximum(cnt, 1.0)
    pn = jnp.sqrt(jnp.sum(pool * pool, axis=1, keepdims=True))
    pool_out[...] = pool / jnp.maximum(pn, 1e-12)

    less = (n2g < gids).astype(jnp.float32)
    cntl = lax.dot_general(ones_col, less, cdims, preferred_element_type=jnp.float32)
    aidx = jnp.minimum(cntl, float(_N - 1)).astype(jnp.int32)
    nio = lax.broadcasted_iota(jnp.int32, (_N, _B), 0)
    aoh = (nio == aidx).astype(jnp.float32)
    ax = lax.dot_general(aoh, x_ref[...], cdims, preferred_element_type=jnp.float32)
    ao = jnp.dot(ax, w_ref[...], preferred_element_type=jnp.float32) + bias
    aop = jnp.maximum(ao, 0.0) + a * jnp.minimum(ao, 0.0)
    an = jnp.sqrt(jnp.sum(aop * aop, axis=1, keepdims=True))
    anc_out[...] = aop / jnp.maximum(an, 1e-12)


_epi_call = pl.pallas_call(
    _epi_body,
    out_shape=[
        jax.ShapeDtypeStruct((_N, _DOUT), jnp.float32),
        jax.ShapeDtypeStruct((_B, _DOUT), jnp.float32),
        jax.ShapeDtypeStruct((_B, _DOUT), jnp.float32),
    ],
)


def kernel(x, edge_index, edge_weight, node2graph, W, b, prelu_a):
    src = edge_index[0]
    dst = edge_index[1]
    xw = _mm_call(x, W)
    dego, degi = _deg_kernel(src, dst)
    pad = _EP - _E
    spread = jnp.arange(pad, dtype=jnp.int32) % _N
    srcp = jnp.concatenate([src, spread])
    dstp = jnp.concatenate([dst, spread])
    ewp = jnp.concatenate([edge_weight, jnp.zeros((pad,), jnp.float32)])
    agg2 = _edge_kernel(xw, dego,
                        srcp.reshape(_EP // _CK, _CK),
                        dstp.reshape(_EP // _CK, _CK),
                        ewp.reshape(_EP // _CK, _CK))
    h, pool, anc = _epi_call(x, W, agg2, degi.reshape(_NC, _N, 1),
                             node2graph.reshape(_N, 1),
                             b.reshape(1, _DOUT),
                             jnp.asarray(prelu_a, jnp.float32).reshape(1, 1))
    return h, pool, anc


# speedup vs baseline: 2.3791x; 1.0281x over previous
import functools

import jax
import jax.numpy as jnp
from jax import lax
from jax.experimental import pallas as pl
from jax.experimental.pallas import tpu as pltpu
from jax.experimental.pallas import tpu_sc as plsc

_N = 10000
_E = 320000
_DIN = 128
_DOUT = 64
_B = 64

_NC = 2
_NS = 16
_NW = _NC * _NS
_EPW = _E // _NW
_CK = 256
_NCH = 40
_EP = _NW * _CK * _NCH
_RPT = 632
_SEGS = ((0, 248), (248, 248), (496, 136))

_mesh = plsc.VectorSubcoreMesh(core_axis_name="c", subcore_axis_name="s")
_sc_params = pltpu.CompilerParams(use_tc_tiling_on_sc=False,
                                  needs_layout_passes=False)


@functools.partial(
    pl.kernel,
    out_type=[
        jax.ShapeDtypeStruct((_NC, _N), jnp.float32),
        jax.ShapeDtypeStruct((_NC, _N), jnp.float32),
    ],
    mesh=_mesh,
    scratch_types=[
        pltpu.VMEM((_EPW,), jnp.int32),
        pltpu.VMEM((_EPW,), jnp.int32),
        pltpu.VMEM((_EPW,), jnp.float32),
        pltpu.VMEM_SHARED((_N,), jnp.float32),
        pltpu.VMEM_SHARED((_N,), jnp.float32),
        pltpu.SemaphoreType.DMA,
        pltpu.SemaphoreType.DMA,
    ],
)
def _deg_kernel(src_hbm, dst_hbm, dego_hbm, degi_hbm,
                srcv, dstv, onesv, dego_s, degi_s, sem0, sem1):
    cid = lax.axis_index("c")
    sid = lax.axis_index("s")
    base = (cid * _NS + sid) * _EPW
    pltpu.async_copy(src_hbm.at[pl.ds(base, _EPW)], srcv, sem0)
    pltpu.async_copy(dst_hbm.at[pl.ds(base, _EPW)], dstv, sem1)

    @plsc.parallel_loop(0, _EPW // 16, unroll=8)
    def zloop(i):
        onesv[pl.ds(i * 16, 16)] = jnp.zeros((16,), jnp.float32)

    @pl.when(sid == 0)
    def _():
        pltpu.sync_copy(onesv, dego_s)
        pltpu.sync_copy(onesv, degi_s)

    @plsc.parallel_loop(0, _EPW // 16, unroll=8)
    def oloop(i):
        onesv[pl.ds(i * 16, 16)] = jnp.ones((16,), jnp.float32)

    plsc.subcore_barrier()
    pltpu.make_async_copy(src_hbm.at[pl.ds(base, _EPW)], srcv, sem0).wait()
    pltpu.make_async_copy(dst_hbm.at[pl.ds(base, _EPW)], dstv, sem1).wait()
    pltpu.async_copy(onesv, dego_s.at[srcv], sem0, add=True)
    pltpu.async_copy(onesv, degi_s.at[dstv], sem1, add=True)
    pltpu.make_async_copy(onesv, dego_s.at[srcv], sem0).wait()
    pltpu.make_async_copy(onesv, degi_s.at[dstv], sem1).wait()
    plsc.subcore_barrier()

    @pl.when(sid == 0)
    def _():
        pltpu.sync_copy(dego_s, dego_hbm.at[cid])
        pltpu.sync_copy(degi_s, degi_hbm.at[cid])


@functools.partial(
    pl.kernel,
    out_type=jax.ShapeDtypeStruct((_NC * _N, _DOUT), jnp.float32),
    mesh=_mesh,
    scratch_types=[
        pltpu.VMEM((_NCH, _CK), jnp.int32),
        pltpu.VMEM((_NCH, _CK), jnp.int32),
        pltpu.VMEM((_NCH, _CK), jnp.float32),
        pltpu.VMEM((_CK, _DOUT), jnp.float32),
        pltpu.VMEM((_CK, _DOUT), jnp.float32),
        pltpu.VMEM((_N,), jnp.float32),
        pltpu.VMEM((640,), jnp.float32),
        pltpu.VMEM((640,), jnp.float32),
        pltpu.VMEM((_CK,), jnp.float32),
        pltpu.VMEM_SHARED((_N,), jnp.float32),
        pltpu.VMEM_SHARED((_N, _DOUT), jnp.float32),
        pltpu.SemaphoreType.DMA,
        pltpu.SemaphoreType.DMA,
        pltpu.SemaphoreType.DMA,
        pltpu.SemaphoreType.DMA,
    ],
    compiler_params=_sc_params,
)
def _edge_kernel(xw_hbm, dego_hbm, src_hbm, dst_hbm, ew_hbm, agg_hbm,
                 srcb, dstb, ewb, rows0, rows1, normv, na, nb, wbuf,
                 norm_s, agg_s, semg0, semg1, sems0, sems1):
    cid = lax.axis_index("c")
    sid = lax.axis_index("s")
    r0 = (cid * _NS + sid) * _NCH
    pltpu.sync_copy(src_hbm.at[pl.ds(r0, _NCH)], srcb)
    pltpu.sync_copy(dst_hbm.at[pl.ds(r0, _NCH)], dstb)
    pltpu.sync_copy(ew_hbm.at[pl.ds(r0, _NCH)], ewb)

    @plsc.parallel_loop(0, _CK, unroll=8)
    def zloop(i):
        for c in range(_DOUT // 16):
            rows0[i, pl.ds(c * 16, 16)] = jnp.zeros((16,), jnp.float32)
    row0 = pl.multiple_of(jnp.minimum(sid * _RPT, _N - _RPT), 8)
    for o, ln in _SEGS:
        pltpu.sync_copy(rows0.at[pl.ds(0, ln)], agg_s.at[pl.ds(row0 + o, ln)])
    pltpu.async_copy(xw_hbm.at[srcb.at[0]], rows0, semg0)

    offn = pl.multiple_of(sid * 624, 16)
    pltpu.sync_copy(dego_hbm.at[0, pl.ds(offn, 640)], na)
    pltpu.sync_copy(dego_hbm.at[1, pl.ds(offn, 640)], nb)

    @plsc.parallel_loop(0, 40, unroll=4)
    def nloop(q):
        sl = pl.ds(q * 16, 16)
        x = jnp.maximum(na[sl] + nb[sl], 1.0)
        i32 = plsc.bitcast(x, jnp.int32)
        y = plsc.bitcast(
            jnp.full((16,), 0x5F3759DF, jnp.int32)
            - lax.shift_right_logical(i32, 1), jnp.float32)
        hx = 0.5 * x
        for _it in range(3):
            y = y * (1.5 - hx * y * y)
        na[sl] = y
    pltpu.sync_copy(na, norm_s.at[pl.ds(offn, 640)])
    plsc.subcore_barrier()
    pltpu.sync_copy(norm_s, normv)

    def scale(rows, j):
        jv = jnp.full((16,), j, jnp.int32)

        @plsc.parallel_loop(0, _CK // 16, unroll=4)
        def wloop(q):
            sl = pl.ds(q * 16, 16)
            sv = srcb[j, sl]
            nv = plsc.load_gather(normv, [sv])
            wbuf[sl] = ewb[j, sl] * nv

        @plsc.parallel_loop(0, _CK, unroll=8)
        def srow(i):
            wv = plsc.load_gather(wbuf, [jnp.full((16,), i, jnp.int32)])
            for c in range(_DOUT // 16):
                sl = pl.ds(c * 16, 16)
                rows[i, sl] = rows[i, sl] * wv

    def body(g, _):
        c0 = 2 * g
        c1 = c0 + 1
        @pl.when(g > 0)
        def _():
            pltpu.make_async_copy(rows1, agg_s.at[dstb.at[c1]], sems1).wait()
        pltpu.async_copy(xw_hbm.at[srcb.at[c1]], rows1, semg1)
        pltpu.make_async_copy(xw_hbm.at[srcb.at[c0]], rows0, semg0).wait()
        scale(rows0, c0)
        pltpu.async_copy(rows0, agg_s.at[dstb.at[c0]], sems0, add=True)
        pltpu.make_async_copy(xw_hbm.at[srcb.at[c1]], rows1, semg1).wait()
        scale(rows1, c1)
        pltpu.async_copy(rows1, agg_s.at[dstb.at[c1]], sems1, add=True)
        pltpu.make_async_copy(rows0, agg_s.at[dstb.at[c0]], sems0).wait()
        @pl.when(g < _NCH // 2 - 1)
        def _():
            pltpu.async_copy(xw_hbm.at[srcb.at[c0 + 2]], rows0, semg0)
        return 0
    lax.fori_loop(0, _NCH // 2, body, 0)
    pltpu.make_async_copy(rows1, agg_s.at[dstb.at[_NCH - 1]], sems1).wait()

    plsc.subcore_barrier()
    for o, ln in _SEGS:
        pltpu.sync_copy(agg_s.at[pl.ds(row0 + o, ln)],
                        agg_hbm.at[pl.ds(cid * _N + row0 + o, ln)])


def _mm_body(x_ref, w_ref, out_ref):
    out_ref[...] = jnp.dot(x_ref[...], w_ref[...],
                           preferred_element_type=jnp.float32)


_mm_call = pl.pallas_call(
    _mm_body,
    out_shape=jax.ShapeDtypeStruct((_N, _DOUT), jnp.float32),
)


def _epi_body(x_ref, w_ref, agg_ref, degi_ref, n2g_ref, b_ref, a_ref,
              h_out, pool_out, anc_out):
    a = a_ref[0, 0]
    bias = b_ref[...]
    agg = agg_ref[pl.ds(0, _N), :] + agg_ref[pl.ds(_N, _N), :]
    degi = degi_ref[0] + degi_ref[1]
    ndst = lax.rsqrt(jnp.maximum(degi, 1.0))
    h = agg * ndst + bias
    hp = jnp.maximum(h, 0.0) + a * jnp.minimum(h, 0.0)
    hn = jnp.sqrt(jnp.sum(hp * hp, axis=1, keepdims=True))
    h_out[...] = hp / jnp.maximum(hn, 1e-12)

    n2g = n2g_ref[...]
    gids = lax.broadcasted_iota(jnp.int32, (_N, _B), 1)
    oh = (n2g == gids).astype(jnp.float32)
    ones_col = jnp.ones((_N, 1), jnp.float32)
    cdims = (((0,), (0,)), ((), ()))
    pool_sum = lax.dot_general(oh, hp, cdims, preferred_element_type=jnp.float32)
    cnt = lax.dot_general(oh, ones_col, cdims, preferred_element_type=jnp.float32)
    pool = pool_sum / jnp.maximum(cnt, 1.0)
    pn = jnp.sqrt(jnp.sum(pool * pool, axis=1, keepdims=True))
    pool_out[...] = pool / jnp.maximum(pn, 1e-12)

    less = (n2g < gids).astype(jnp.float32)
    cntl = lax.dot_general(ones_col, less, cdims, preferred_element_type=jnp.float32)
    aidx = jnp.minimum(cntl, float(_N - 1)).astype(jnp.int32)
    nio = lax.broadcasted_iota(jnp.int32, (_N, _B), 0)
    aoh = (nio == aidx).astype(jnp.float32)
    ax = lax.dot_general(aoh, x_ref[...], cdims, preferred_element_type=jnp.float32)
    ao = jnp.dot(ax, w_ref[...], preferred_element_type=jnp.float32) + bias
    aop = jnp.maximum(ao, 0.0) + a * jnp.minimum(ao, 0.0)
    an = jnp.sqrt(jnp.sum(aop * aop, axis=1, keepdims=True))
    anc_out[...] = aop / jnp.maximum(an, 1e-12)


_epi_call = pl.pallas_call(
    _epi_body,
    out_shape=[
        jax.ShapeDtypeStruct((_N, _DOUT), jnp.float32),
        jax.ShapeDtypeStruct((_B, _DOUT), jnp.float32),
        jax.ShapeDtypeStruct((_B, _DOUT), jnp.float32),
    ],
)


def kernel(x, edge_index, edge_weight, node2graph, W, b, prelu_a):
    src = edge_index[0]
    dst = edge_index[1]
    xw = _mm_call(x, W)
    dego, degi = _deg_kernel(src, dst)
    pad = _EP - _E
    spread = jnp.arange(pad, dtype=jnp.int32) % _N
    srcp = jnp.concatenate([src, spread])
    dstp = jnp.concatenate([dst, spread])
    ewp = jnp.concatenate([edge_weight, jnp.zeros((pad,), jnp.float32)])
    agg2 = _edge_kernel(xw, dego,
                        srcp.reshape(_EP // _CK, _CK),
                        dstp.reshape(_EP // _CK, _CK),
                        ewp.reshape(_EP // _CK, _CK))
    h, pool, anc = _epi_call(x, W, agg2, degi.reshape(_NC, _N, 1),
                             node2graph.reshape(_N, 1),
                             b.reshape(1, _DOUT),
                             jnp.asarray(prelu_a, jnp.float32).reshape(1, 1))
    return h, pool, anc
